# baseline (speedup 1.0000x reference)
import functools

import jax
import jax.numpy as jnp
from jax import lax
from jax.experimental import pallas as pl
from jax.experimental.pallas import tpu as pltpu
from jax.experimental.pallas import tpu_sc as plsc

Q, C, D = 4096, 32, 64
NC, NS, L = 2, 16, 16
NW = NC * NS
QPB = 4
NBT = Q // QPB
NB = NBT // NW
BR = QPB * C
QB = 64
RB = QB * (C // 2)
INVERSE_SIGMA = 10.0

_mesh = plsc.VectorSubcoreMesh(core_axis_name="c", subcore_axis_name="s")


@functools.partial(
    pl.kernel,
    mesh=_mesh,
    compiler_params=pltpu.CompilerParams(
        needs_layout_passes=False, use_tc_tiling_on_sc=False),
    out_type=jax.ShapeDtypeStruct((Q * C, D), jnp.float32),
    scratch_types=[
        pltpu.VMEM((NB, BR), jnp.int32),
        pltpu.VMEM((2, BR, D), jnp.float32),
        pltpu.SemaphoreType.DMA((2,)),
    ],
)
def _sc_gather(k_hbm, X_hbm, out_hbm, k_v, rows_v, sem):
    wid = lax.axis_index("s") * NC + lax.axis_index("c")
    pltpu.sync_copy(k_hbm.at[pl.ds(wid * NB, NB)], k_v)

    def gather(b, slot):
        return pltpu.make_async_copy(
            X_hbm.at[k_v.at[b]], rows_v.at[slot], sem.at[slot])

    gather(0, 0).start()

    def body(i, carry):
        slot = lax.rem(i, 2)
        nxt = lax.rem(i + 1, 2)
        gather(jnp.minimum(i + 1, NB - 1), nxt).start()
        gather(i, slot).wait()
        row0 = (wid * NB + i) * BR
        pltpu.sync_copy(rows_v.at[slot], out_hbm.at[pl.ds(row0, BR)])
        return carry

    lax.fori_loop(0, NB, body, 0, unroll=False)
    gather(NB - 1, lax.rem(NB, 2)).wait()


def _expm1s(z):
    return jnp.where(jnp.abs(z) < 1e-3, z + 0.5 * z * z, jnp.exp(z) - 1.0)


def _tc_body(xk_ref, x_ref, v_ref, o_ref):
    f32 = jnp.float32
    xk = xk_ref[...]
    x2 = x_ref[...]
    v2 = v_ref[...]
    H = C // 2

    def rows(a):
        w = a.shape[-1]
        return jnp.reshape(jnp.broadcast_to(a[:, None, :], (QB, H, w)), (RB, w))

    delta = xk - rows(x2)
    prod = delta * rows(v2)
    d2 = delta * delta
    lane = lax.broadcasted_iota(jnp.int32, (RB, 128), 1)
    left = lane < D
    dl = jnp.sum(jnp.where(left, prod, 0.0), axis=1, keepdims=True)
    dr = jnp.sum(jnp.where(left, 0.0, prod), axis=1, keepdims=True)
    nl = jnp.sum(jnp.where(left, d2, 0.0), axis=1, keepdims=True)
    nr = jnp.sum(jnp.where(left, 0.0, d2), axis=1, keepdims=True)
    laneq = lax.broadcasted_iota(jnp.int32, (QB, 128), 1)
    nv2 = jnp.sum(jnp.where(laneq < D, v2 * v2, 0.0), axis=1, keepdims=True)
    nv2e = rows(nv2)
    sl = dl / jnp.maximum(jnp.sqrt(nl * nv2e), 1e-8)
    sr = dr / jnp.maximum(jnp.sqrt(nr * nv2e), 1e-8)
    tl = _expm1s(INVERSE_SIGMA * sl)
    tr = _expm1s(INVERSE_SIGMA * sr)
    seg = (lax.broadcasted_iota(jnp.int32, (QB, RB), 1) // H
           == lax.broadcasted_iota(jnp.int32, (QB, RB), 0)).astype(f32)
    mm = lambda a, b: jax.lax.dot(a, b, precision=jax.lax.Precision.HIGHEST,
                                  preferred_element_type=f32)
    mean = mm(seg, tl + tr) * (1.0 / C)
    invs = 1.0 / mm(seg, jnp.abs(tl) + jnp.abs(tr))
    wl = (tl - rows(mean)) * rows(invs)
    wr = (tr - rows(mean)) * rows(invs)
    wfull = jnp.where(left, wl, wr)
    res = mm(seg, xk * wfull)
    o_ref[...] = res[:, :D] + res[:, D:]


def _tc_compute(xk2, x2, v2):
    return pl.pallas_call(
        _tc_body,
        grid=(Q // QB,),
        in_specs=[
            pl.BlockSpec((RB, 2 * D), lambda b: (b, 0)),
            pl.BlockSpec((QB, 2 * D), lambda b: (b, 0)),
            pl.BlockSpec((QB, 2 * D), lambda b: (b, 0)),
        ],
        out_specs=pl.BlockSpec((QB, D), lambda b: (b, 0)),
        out_shape=jax.ShapeDtypeStruct((Q, D), jnp.float32),
    )(xk2, x2, v2)


def kernel(x, v, k, X):
    k32 = k.astype(jnp.int32).reshape(NBT, BR)
    xk = _sc_gather(k32, X)
    xk2 = xk.reshape(Q * C // 2, 2 * D)
    x2 = jnp.concatenate([x, x], axis=1)
    v2 = jnp.concatenate([v, v], axis=1)
    return _tc_compute(xk2, x2, v2)
